# Initial kernel scaffold; baseline (speedup 1.0000x reference)
import jax
import jax.numpy as jnp
from jax.experimental import pallas as pl


def kernel(node_type, mesh_pos, world_pos, known_vel, srcs, dsts, wsrcs, wdsts, params):
    raise NotImplementedError("write your pallas kernel here")


# speedup vs baseline: 1.1030x; 1.1030x over previous
import functools
import math

import jax
import jax.numpy as jnp
from jax import lax
from jax.experimental import pallas as pl
from jax.experimental.pallas import tpu as pltpu

N = 10000
EM = 160000
EW = 40000
EWP = 40960
LATENT = 128
NUM_MP = 15

F32 = jnp.float32


def _ln(x, scale, bias):
    m = x.mean(-1, keepdims=True)
    v = ((x - m) ** 2).mean(-1, keepdims=True)
    return (x - m) / jnp.sqrt(v + 1e-5) * scale + bias


def _dot(a, b):
    return jnp.dot(a, b, preferred_element_type=F32)


def _node_enc_body(nt_ref, kv_ref, w1_ref, b1_ref, w2_ref, b2_ref,
                   w3_ref, b3_ref, ls_ref, lb_ref, out_ref):
    blk = nt_ref.shape[0]
    t = nt_ref[...]
    iota = lax.broadcasted_iota(jnp.int32, (blk, 16), 1)
    oh = (iota == (t + 3)).astype(F32)
    kv = kv_ref[...]
    col = lax.broadcasted_iota(jnp.int32, (blk, 16), 1)
    kv16 = jnp.concatenate([kv, jnp.zeros((blk, 8), F32)], axis=1)
    nf = jnp.where(col < 3, kv16, oh)
    x = jnp.maximum(_dot(nf, w1_ref[...]) + b1_ref[...], 0.0)
    x = jnp.maximum(_dot(x, w2_ref[...]) + b2_ref[...], 0.0)
    x = _dot(x, w3_ref[...]) + b3_ref[...]
    out_ref[...] = _ln(x, ls_ref[...], lb_ref[...])


def _node_encoder(node_type, known_vel, w1, b1, w2, b2, w3, b3, ls, lb):
    blk = 2000
    grid = N // blk
    full = lambda shape: pl.BlockSpec(shape, lambda i: (0, 0))
    return pl.pallas_call(
        _node_enc_body,
        grid=(grid,),
        in_specs=[
            pl.BlockSpec((blk, 1), lambda i: (i, 0)),
            pl.BlockSpec((blk, 8), lambda i: (i, 0)),
            full((16, LATENT)), full((1, LATENT)),
            full((LATENT, LATENT)), full((1, LATENT)),
            full((LATENT, LATENT)), full((1, LATENT)),
            full((1, LATENT)), full((1, LATENT)),
        ],
        out_specs=pl.BlockSpec((blk, LATENT), lambda i: (i, 0)),
        out_shape=jax.ShapeDtypeStruct((N, LATENT), F32),
    )(node_type, known_vel, w1, b1, w2, b2, w3, b3, ls, lb)


def _edge_enc_body(rel_ref, w1_ref, wn1_ref, wn2_ref, b1_ref, w2_ref, b2_ref,
                   w3_ref, b3_ref, ls_ref, lb_ref, out_ref, *, two_norms,
                   valid_rows, blk):
    r = rel_ref[...]
    sq = r * r
    n1 = jnp.sqrt(sq[:, 0:1] + sq[:, 1:2] + sq[:, 2:3])
    x = _dot(r, w1_ref[...]) + n1 * wn1_ref[...] + b1_ref[...]
    if two_norms:
        n2 = jnp.sqrt(sq[:, 3:4] + sq[:, 4:5] + sq[:, 5:6])
        x = x + n2 * wn2_ref[...]
    x = jnp.maximum(x, 0.0)
    x = jnp.maximum(_dot(x, w2_ref[...]) + b2_ref[...], 0.0)
    x = _dot(x, w3_ref[...]) + b3_ref[...]
    x = _ln(x, ls_ref[...], lb_ref[...])
    if valid_rows is not None:
        row = pl.program_id(0) * blk + lax.broadcasted_iota(jnp.int32, x.shape, 0)
        x = jnp.where(row < valid_rows, x, 0.0)
    out_ref[...] = x


def _edge_encoder(rel8, w1, wn1, wn2, b1, w2, b2, w3, b3, ls, lb, *,
                  two_norms, valid_rows=None):
    e = rel8.shape[0]
    blk = 2048 if e % 2048 == 0 else 2000
    grid = e // blk
    full = lambda shape: pl.BlockSpec(shape, lambda i: (0, 0))
    body = functools.partial(_edge_enc_body, two_norms=two_norms,
                             valid_rows=valid_rows, blk=blk)
    return pl.pallas_call(
        body,
        grid=(grid,),
        in_specs=[
            pl.BlockSpec((blk, 8), lambda i: (i, 0)),
            full((8, LATENT)), full((1, LATENT)), full((1, LATENT)),
            full((1, LATENT)),
            full((LATENT, LATENT)), full((1, LATENT)),
            full((LATENT, LATENT)), full((1, LATENT)),
            full((1, LATENT)), full((1, LATENT)),
        ],
        out_specs=pl.BlockSpec((blk, LATENT), lambda i: (i, 0)),
        out_shape=jax.ShapeDtypeStruct((e, LATENT), F32),
    )(rel8, w1, wn1, wn2, b1, w2, b2, w3, b3, ls, lb)


def _edge_mp_body(e_ref, g_ref, w1_ref, b1_ref, w2_ref, b2_ref, w3_ref,
                  b3_ref, ls_ref, lb_ref, out_ref, *, valid_rows, blk):
    e = e_ref[...]
    x = jnp.maximum(_dot(e, w1_ref[...]) + g_ref[...] + b1_ref[...], 0.0)
    x = jnp.maximum(_dot(x, w2_ref[...]) + b2_ref[...], 0.0)
    x = _dot(x, w3_ref[...]) + b3_ref[...]
    x = e + _ln(x, ls_ref[...], lb_ref[...])
    if valid_rows is not None:
        row = pl.program_id(0) * blk + lax.broadcasted_iota(jnp.int32, x.shape, 0)
        x = jnp.where(row < valid_rows, x, 0.0)
    out_ref[...] = x


def _edge_mp(e, g, w1, b1, w2, b2, w3, b3, ls, lb, *, valid_rows=None):
    ne = e.shape[0]
    blk = 2048 if ne % 2048 == 0 else 2000
    grid = ne // blk
    full = lambda shape: pl.BlockSpec(shape, lambda i: (0, 0))
    body = functools.partial(_edge_mp_body, valid_rows=valid_rows, blk=blk)
    return pl.pallas_call(
        body,
        grid=(grid,),
        in_specs=[
            pl.BlockSpec((blk, LATENT), lambda i: (i, 0)),
            pl.BlockSpec((blk, LATENT), lambda i: (i, 0)),
            full((LATENT, LATENT)), full((1, LATENT)),
            full((LATENT, LATENT)), full((1, LATENT)),
            full((LATENT, LATENT)), full((1, LATENT)),
            full((1, LATENT)), full((1, LATENT)),
        ],
        out_specs=pl.BlockSpec((blk, LATENT), lambda i: (i, 0)),
        out_shape=jax.ShapeDtypeStruct((ne, LATENT), F32),
    )(e, g, w1, b1, w2, b2, w3, b3, ls, lb)


def _node_mp_body(lat_ref, a0_ref, a1_ref, w1a_ref, w1b_ref, w1c_ref, b1_ref,
                  w2_ref, b2_ref, w3_ref, b3_ref, ls_ref, lb_ref, vcat_ref,
                  out_ref, p_ref, *, nv):
    latv = lat_ref[...]
    x = (_dot(latv, w1a_ref[...]) + _dot(a0_ref[...], w1b_ref[...])
         + _dot(a1_ref[...], w1c_ref[...]) + b1_ref[...])
    x = jnp.maximum(x, 0.0)
    x = jnp.maximum(_dot(x, w2_ref[...]) + b2_ref[...], 0.0)
    x = _dot(x, w3_ref[...]) + b3_ref[...]
    new_lat = latv + _ln(x, ls_ref[...], lb_ref[...])
    out_ref[...] = new_lat
    if nv:
        p_ref[...] = _dot(new_lat, vcat_ref[...])


def _node_mp(lat, a0, a1, w1a, w1b, w1c, b1, w2, b2, w3, b3, ls, lb, vcat):
    blk = 2000
    grid = N // blk
    nv = vcat.shape[1]
    full = lambda shape: pl.BlockSpec(shape, lambda i: (0, 0))
    body = functools.partial(_node_mp_body, nv=nv)
    return pl.pallas_call(
        body,
        grid=(grid,),
        in_specs=[
            pl.BlockSpec((blk, LATENT), lambda i: (i, 0)),
            pl.BlockSpec((blk, LATENT), lambda i: (i, 0)),
            pl.BlockSpec((blk, LATENT), lambda i: (i, 0)),
            full((LATENT, LATENT)), full((LATENT, LATENT)),
            full((LATENT, LATENT)), full((1, LATENT)),
            full((LATENT, LATENT)), full((1, LATENT)),
            full((LATENT, LATENT)), full((1, LATENT)),
            full((1, LATENT)), full((1, LATENT)),
            full((LATENT, max(nv, LATENT))),
        ],
        out_specs=[
            pl.BlockSpec((blk, LATENT), lambda i: (i, 0)),
            pl.BlockSpec((blk, max(nv, LATENT)), lambda i: (i, 0)),
        ],
        out_shape=[
            jax.ShapeDtypeStruct((N, LATENT), F32),
            jax.ShapeDtypeStruct((N, max(nv, LATENT)), F32),
        ],
    )(lat, a0, a1, w1a, w1b, w1c, b1, w2, b2, w3, b3, ls, lb, vcat)


def _precompute_body(lat_ref, vcat_ref, p_ref):
    p_ref[...] = _dot(lat_ref[...], vcat_ref[...])


def _precompute(lat, vcat):
    blk = 2000
    grid = N // blk
    nv = vcat.shape[1]
    return pl.pallas_call(
        _precompute_body,
        grid=(grid,),
        in_specs=[
            pl.BlockSpec((blk, LATENT), lambda i: (i, 0)),
            pl.BlockSpec((LATENT, nv), lambda i: (0, 0)),
        ],
        out_specs=pl.BlockSpec((blk, nv), lambda i: (i, 0)),
        out_shape=jax.ShapeDtypeStruct((N, nv), F32),
    )(lat, vcat)


def _decoder_body(lat_ref, w1_ref, b1_ref, w2_ref, b2_ref, w3_ref, b3_ref,
                  out_ref):
    x = jnp.maximum(_dot(lat_ref[...], w1_ref[...]) + b1_ref[...], 0.0)
    x = jnp.maximum(_dot(x, w2_ref[...]) + b2_ref[...], 0.0)
    out_ref[...] = _dot(x, w3_ref[...]) + b3_ref[...]


def _decoder(lat, w1, b1, w2, b2, w3, b3):
    blk = 2000
    grid = N // blk
    full = lambda shape: pl.BlockSpec(shape, lambda i: (0, 0))
    return pl.pallas_call(
        _decoder_body,
        grid=(grid,),
        in_specs=[
            pl.BlockSpec((blk, LATENT), lambda i: (i, 0)),
            full((LATENT, LATENT)), full((1, LATENT)),
            full((LATENT, LATENT)), full((1, LATENT)),
            full((LATENT, LATENT)), full((1, LATENT)),
        ],
        out_specs=pl.BlockSpec((blk, LATENT), lambda i: (i, 0)),
        out_shape=jax.ShapeDtypeStruct((N, LATENT), F32),
    )(lat, w1, b1, w2, b2, w3, b3)


def _gather_add(ps, pd, idx_s, idx_d):
    return ps[idx_s] + pd[idx_d]


def _segment_sum(vals, idx):
    return jax.ops.segment_sum(vals, idx, num_segments=N)


def _row(b):
    return b.reshape(1, -1)


def _fold_norm(w1, b1, mean, std):
    wn = w1 / std[:, None]
    b1f = b1 - (mean / std) @ w1
    return wn, b1f


def kernel(node_type, mesh_pos, world_pos, known_vel, srcs, dsts, wsrcs,
           wdsts, params):
    p = params

    n_w1, n_b1 = _fold_norm(p['node_enc']['W'][0], p['node_enc']['b'][0],
                            p['node_mean'], p['node_std'])
    n_w1p = jnp.zeros((16, LATENT), F32).at[:12].set(n_w1)

    m_w1, m_b1 = _fold_norm(p['mesh_enc']['W'][0], p['mesh_enc']['b'][0],
                            p['mesh_mean'], p['mesh_std'])
    m_w1p = jnp.zeros((8, LATENT), F32)
    m_w1p = m_w1p.at[0:3].set(m_w1[0:3]).at[3:6].set(m_w1[4:7])
    m_wn1 = _row(m_w1[3])
    m_wn2 = _row(m_w1[7])

    w_w1, w_b1 = _fold_norm(p['world_enc']['W'][0], p['world_enc']['b'][0],
                            p['world_mean'], p['world_std'])
    w_w1p = jnp.zeros((8, LATENT), F32).at[0:3].set(w_w1[0:3])
    w_wn1 = _row(w_w1[3])

    d_w3 = p['decoder']['W'][2] * p['out_std'][None, :]
    d_b3 = p['decoder']['b'][2] * p['out_std'] + p['out_mean']
    d_w3p = jnp.zeros((LATENT, LATENT), F32).at[:, :3].set(d_w3)
    d_b3p = jnp.zeros((LATENT,), F32).at[:3].set(d_b3)

    wsrcs_p = jnp.concatenate([wsrcs, jnp.zeros((EWP - EW,), jnp.int32)])
    wdsts_p = jnp.concatenate([wdsts, jnp.zeros((EWP - EW,), jnp.int32)])

    pos_src = jnp.concatenate(
        [mesh_pos, world_pos, jnp.zeros((N, 2), F32)], axis=1)
    pos_dst = jnp.concatenate(
        [mesh_pos, mesh_pos, jnp.zeros((N, 2), F32)], axis=1)
    pos_w = jnp.concatenate(
        [world_pos, jnp.zeros((N, 5), F32)], axis=1)
    rel8_m = pos_src[srcs] - pos_dst[dsts]
    rel8_w = pos_w[wsrcs_p] - pos_w[wdsts_p]

    ne = p['node_enc']
    lat = _node_encoder(
        node_type.reshape(N, 1),
        jnp.concatenate([known_vel, jnp.zeros((N, 5), F32)], axis=1),
        n_w1p, _row(n_b1), ne['W'][1], _row(ne['b'][1]), ne['W'][2],
        _row(ne['b'][2]), _row(ne['ln_scale']), _row(ne['ln_bias']))

    me = p['mesh_enc']
    e0 = _edge_encoder(rel8_m, m_w1p, m_wn1, m_wn2, _row(m_b1), me['W'][1],
                       _row(me['b'][1]), me['W'][2], _row(me['b'][2]),
                       _row(me['ln_scale']), _row(me['ln_bias']),
                       two_norms=True)
    we = p['world_enc']
    e1 = _edge_encoder(rel8_w, w_w1p, w_wn1, w_wn1, _row(w_b1), we['W'][1],
                       _row(we['b'][1]), we['W'][2], _row(we['b'][2]),
                       _row(we['ln_scale']), _row(we['ln_bias']),
                       two_norms=False, valid_rows=EW)

    def vcat_for(step):
        em, ew = p['edge_mp'][step]
        return jnp.concatenate(
            [em['W'][0][LATENT:2 * LATENT], em['W'][0][2 * LATENT:],
             ew['W'][0][LATENT:2 * LATENT], ew['W'][0][2 * LATENT:]], axis=1)

    ptab = _precompute(lat, vcat_for(0))
    for step in range(NUM_MP):
        em, ew = p['edge_mp'][step]
        nmp = p['node_mp'][step]
        psm, pdm = ptab[:, 0:LATENT], ptab[:, LATENT:2 * LATENT]
        psw, pdw = ptab[:, 2 * LATENT:3 * LATENT], ptab[:, 3 * LATENT:]

        gm = _gather_add(psm, pdm, srcs, dsts)
        gw = _gather_add(psw, pdw, wsrcs_p, wdsts_p)

        ne0 = _edge_mp(e0, gm, em['W'][0][:LATENT], _row(em['b'][0]),
                       em['W'][1], _row(em['b'][1]), em['W'][2],
                       _row(em['b'][2]), _row(em['ln_scale']),
                       _row(em['ln_bias']))
        ne1 = _edge_mp(e1, gw, ew['W'][0][:LATENT], _row(ew['b'][0]),
                       ew['W'][1], _row(ew['b'][1]), ew['W'][2],
                       _row(ew['b'][2]), _row(ew['ln_scale']),
                       _row(ew['ln_bias']), valid_rows=EW)

        agg0 = _segment_sum(ne0, dsts)
        agg1 = _segment_sum(ne1, wdsts_p)

        vcat = vcat_for(step + 1) if step + 1 < NUM_MP else jnp.zeros((LATENT, 0), F32)
        w1 = nmp['W'][0]
        lat, ptab = _node_mp(
            lat, agg0, agg1, w1[:LATENT], w1[LATENT:2 * LATENT],
            w1[2 * LATENT:], _row(nmp['b'][0]), nmp['W'][1],
            _row(nmp['b'][1]), nmp['W'][2], _row(nmp['b'][2]),
            _row(nmp['ln_scale']), _row(nmp['ln_bias']),
            jnp.zeros((LATENT, LATENT), F32) if vcat.shape[1] == 0 else vcat)
        e0, e1 = ne0, ne1

    dec = p['decoder']
    out = _decoder(lat, dec['W'][0], _row(dec['b'][0]), dec['W'][1],
                   _row(dec['b'][1]), d_w3p, _row(d_b3p))
    return out[:, :3]


# speedup vs baseline: 1.8144x; 1.6450x over previous
import functools
import math

import jax
import jax.numpy as jnp
from jax import lax
from jax.experimental import pallas as pl
from jax.experimental.pallas import tpu as pltpu
from jax.experimental.pallas import tpu_sc as plsc

NC = 2
NS = 16
NW = NC * NS

N = 10000
NP = 10240
EM = 160000
EMP = 163840
EW = 40000
EWP = 40960
LATENT = 128
NUM_MP = 15

F32 = jnp.float32


def _ln(x, scale, bias):
    m = x.mean(-1, keepdims=True)
    v = ((x - m) ** 2).mean(-1, keepdims=True)
    return (x - m) / jnp.sqrt(v + 1e-5) * scale + bias


def _dot(a, b):
    return jnp.dot(a, b, preferred_element_type=F32)


def _node_enc_body(nt_ref, kv_ref, w1_ref, b1_ref, w2_ref, b2_ref,
                   w3_ref, b3_ref, ls_ref, lb_ref, out_ref):
    blk = nt_ref.shape[0]
    t = nt_ref[...]
    iota = lax.broadcasted_iota(jnp.int32, (blk, 16), 1)
    oh = (iota == (t + 3)).astype(F32)
    kv = kv_ref[...]
    col = lax.broadcasted_iota(jnp.int32, (blk, 16), 1)
    kv16 = jnp.concatenate([kv, jnp.zeros((blk, 8), F32)], axis=1)
    nf = jnp.where(col < 3, kv16, oh)
    x = jnp.maximum(_dot(nf, w1_ref[...]) + b1_ref[...], 0.0)
    x = jnp.maximum(_dot(x, w2_ref[...]) + b2_ref[...], 0.0)
    x = _dot(x, w3_ref[...]) + b3_ref[...]
    out_ref[...] = _ln(x, ls_ref[...], lb_ref[...])


def _node_encoder(node_type, known_vel, w1, b1, w2, b2, w3, b3, ls, lb):
    blk = 2048
    grid = NP // blk
    full = lambda shape: pl.BlockSpec(shape, lambda i: (0, 0))
    return pl.pallas_call(
        _node_enc_body,
        grid=(grid,),
        in_specs=[
            pl.BlockSpec((blk, 1), lambda i: (i, 0)),
            pl.BlockSpec((blk, 8), lambda i: (i, 0)),
            full((16, LATENT)), full((1, LATENT)),
            full((LATENT, LATENT)), full((1, LATENT)),
            full((LATENT, LATENT)), full((1, LATENT)),
            full((1, LATENT)), full((1, LATENT)),
        ],
        out_specs=pl.BlockSpec((blk, LATENT), lambda i: (i, 0)),
        out_shape=jax.ShapeDtypeStruct((NP, LATENT), F32),
    )(node_type, known_vel, w1, b1, w2, b2, w3, b3, ls, lb)


def _edge_enc_body(rel_ref, w1_ref, wn1_ref, wn2_ref, b1_ref, w2_ref, b2_ref,
                   w3_ref, b3_ref, ls_ref, lb_ref, out_ref, *, two_norms,
                   valid_rows, blk):
    r = rel_ref[...]
    sq = r * r
    n1 = jnp.sqrt(sq[:, 0:1] + sq[:, 1:2] + sq[:, 2:3])
    x = _dot(r, w1_ref[...]) + n1 * wn1_ref[...] + b1_ref[...]
    if two_norms:
        n2 = jnp.sqrt(sq[:, 3:4] + sq[:, 4:5] + sq[:, 5:6])
        x = x + n2 * wn2_ref[...]
    x = jnp.maximum(x, 0.0)
    x = jnp.maximum(_dot(x, w2_ref[...]) + b2_ref[...], 0.0)
    x = _dot(x, w3_ref[...]) + b3_ref[...]
    x = _ln(x, ls_ref[...], lb_ref[...])
    if valid_rows is not None:
        row = pl.program_id(0) * blk + lax.broadcasted_iota(jnp.int32, x.shape, 0)
        x = jnp.where(row < valid_rows, x, 0.0)
    out_ref[...] = x


def _edge_encoder(rel8, w1, wn1, wn2, b1, w2, b2, w3, b3, ls, lb, *,
                  two_norms, valid_rows=None):
    e = rel8.shape[0]
    blk = 2048 if e % 2048 == 0 else 2000
    grid = e // blk
    full = lambda shape: pl.BlockSpec(shape, lambda i: (0, 0))
    body = functools.partial(_edge_enc_body, two_norms=two_norms,
                             valid_rows=valid_rows, blk=blk)
    return pl.pallas_call(
        body,
        grid=(grid,),
        in_specs=[
            pl.BlockSpec((blk, 16), lambda i: (i, 0)),
            full((16, LATENT)), full((1, LATENT)), full((1, LATENT)),
            full((1, LATENT)),
            full((LATENT, LATENT)), full((1, LATENT)),
            full((LATENT, LATENT)), full((1, LATENT)),
            full((1, LATENT)), full((1, LATENT)),
        ],
        out_specs=pl.BlockSpec((blk, LATENT), lambda i: (i, 0)),
        out_shape=jax.ShapeDtypeStruct((e, LATENT), F32),
    )(rel8, w1, wn1, wn2, b1, w2, b2, w3, b3, ls, lb)


def _edge_mp_body(e_ref, g_ref, w1_ref, b1_ref, w2_ref, b2_ref, w3_ref,
                  b3_ref, ls_ref, lb_ref, out_ref, *, valid_rows, blk):
    e = e_ref[...]
    x = jnp.maximum(_dot(e, w1_ref[...]) + g_ref[...] + b1_ref[...], 0.0)
    x = jnp.maximum(_dot(x, w2_ref[...]) + b2_ref[...], 0.0)
    x = _dot(x, w3_ref[...]) + b3_ref[...]
    x = e + _ln(x, ls_ref[...], lb_ref[...])
    if valid_rows is not None:
        row = pl.program_id(0) * blk + lax.broadcasted_iota(jnp.int32, x.shape, 0)
        x = jnp.where(row < valid_rows, x, 0.0)
    out_ref[...] = x


def _edge_mp(e, g, w1, b1, w2, b2, w3, b3, ls, lb, *, valid_rows=None):
    ne = e.shape[0]
    blk = 2048 if ne % 2048 == 0 else 2000
    grid = ne // blk
    full = lambda shape: pl.BlockSpec(shape, lambda i: (0, 0))
    body = functools.partial(_edge_mp_body, valid_rows=valid_rows, blk=blk)
    return pl.pallas_call(
        body,
        grid=(grid,),
        in_specs=[
            pl.BlockSpec((blk, LATENT), lambda i: (i, 0)),
            pl.BlockSpec((blk, LATENT), lambda i: (i, 0)),
            full((LATENT, LATENT)), full((1, LATENT)),
            full((LATENT, LATENT)), full((1, LATENT)),
            full((LATENT, LATENT)), full((1, LATENT)),
            full((1, LATENT)), full((1, LATENT)),
        ],
        out_specs=pl.BlockSpec((blk, LATENT), lambda i: (i, 0)),
        out_shape=jax.ShapeDtypeStruct((ne, LATENT), F32),
    )(e, g, w1, b1, w2, b2, w3, b3, ls, lb)


def _node_mp_body(lat_ref, a0a_ref, a0b_ref, a1a_ref, a1b_ref, w1a_ref,
                  w1b_ref, w1c_ref, b1_ref, w2_ref, b2_ref, w3_ref, b3_ref,
                  ls_ref, lb_ref, vcat_ref, out_ref, p0_ref, p1_ref, p2_ref,
                  p3_ref):
    latv = lat_ref[...]
    a0 = a0a_ref[...] + a0b_ref[...]
    a1 = a1a_ref[...] + a1b_ref[...]
    x = (_dot(latv, w1a_ref[...]) + _dot(a0, w1b_ref[...])
         + _dot(a1, w1c_ref[...]) + b1_ref[...])
    x = jnp.maximum(x, 0.0)
    x = jnp.maximum(_dot(x, w2_ref[...]) + b2_ref[...], 0.0)
    x = _dot(x, w3_ref[...]) + b3_ref[...]
    new_lat = latv + _ln(x, ls_ref[...], lb_ref[...])
    out_ref[...] = new_lat
    p = _dot(new_lat, vcat_ref[...])
    p0_ref[...] = p[:, 0:LATENT]
    p1_ref[...] = p[:, LATENT:2 * LATENT]
    p2_ref[...] = p[:, 2 * LATENT:3 * LATENT]
    p3_ref[...] = p[:, 3 * LATENT:]


def _node_mp(lat, a0a, a0b, a1a, a1b, w1a, w1b, w1c, b1, w2, b2, w3, b3,
             ls, lb, vcat):
    blk = 2048
    grid = NP // blk
    full = lambda shape: pl.BlockSpec(shape, lambda i: (0, 0))
    blkspec = pl.BlockSpec((blk, LATENT), lambda i: (i, 0))
    return pl.pallas_call(
        _node_mp_body,
        grid=(grid,),
        in_specs=[
            blkspec, blkspec, blkspec, blkspec, blkspec,
            full((LATENT, LATENT)), full((LATENT, LATENT)),
            full((LATENT, LATENT)), full((1, LATENT)),
            full((LATENT, LATENT)), full((1, LATENT)),
            full((LATENT, LATENT)), full((1, LATENT)),
            full((1, LATENT)), full((1, LATENT)),
            full((LATENT, 4 * LATENT)),
        ],
        out_specs=[blkspec, blkspec, blkspec, blkspec, blkspec],
        out_shape=[jax.ShapeDtypeStruct((NP, LATENT), F32) for _ in range(5)],
    )(lat, a0a, a0b, a1a, a1b, w1a, w1b, w1c, b1, w2, b2, w3, b3, ls, lb,
      vcat)


def _precompute_body(lat_ref, vcat_ref, p0_ref, p1_ref, p2_ref, p3_ref):
    p = _dot(lat_ref[...], vcat_ref[...])
    p0_ref[...] = p[:, 0:LATENT]
    p1_ref[...] = p[:, LATENT:2 * LATENT]
    p2_ref[...] = p[:, 2 * LATENT:3 * LATENT]
    p3_ref[...] = p[:, 3 * LATENT:]


def _precompute(lat, vcat):
    blk = 2048
    grid = NP // blk
    blkspec = pl.BlockSpec((blk, LATENT), lambda i: (i, 0))
    return pl.pallas_call(
        _precompute_body,
        grid=(grid,),
        in_specs=[
            pl.BlockSpec((blk, LATENT), lambda i: (i, 0)),
            pl.BlockSpec((LATENT, 4 * LATENT), lambda i: (0, 0)),
        ],
        out_specs=[blkspec, blkspec, blkspec, blkspec],
        out_shape=[jax.ShapeDtypeStruct((NP, LATENT), F32) for _ in range(4)],
    )(lat, vcat)


def _decoder_body(lat_ref, w1_ref, b1_ref, w2_ref, b2_ref, w3_ref, b3_ref,
                  out_ref):
    x = jnp.maximum(_dot(lat_ref[...], w1_ref[...]) + b1_ref[...], 0.0)
    x = jnp.maximum(_dot(x, w2_ref[...]) + b2_ref[...], 0.0)
    out_ref[...] = _dot(x, w3_ref[...]) + b3_ref[...]


def _decoder(lat, w1, b1, w2, b2, w3, b3):
    blk = 2048
    grid = NP // blk
    full = lambda shape: pl.BlockSpec(shape, lambda i: (0, 0))
    return pl.pallas_call(
        _decoder_body,
        grid=(grid,),
        in_specs=[
            pl.BlockSpec((blk, LATENT), lambda i: (i, 0)),
            full((LATENT, LATENT)), full((1, LATENT)),
            full((LATENT, LATENT)), full((1, LATENT)),
            full((LATENT, LATENT)), full((1, LATENT)),
        ],
        out_specs=pl.BlockSpec((blk, LATENT), lambda i: (i, 0)),
        out_shape=jax.ShapeDtypeStruct((NP, LATENT), F32),
    )(lat, w1, b1, w2, b2, w3, b3)


_MESH = plsc.VectorSubcoreMesh(core_axis_name="c", subcore_axis_name="s",
                               num_cores=NC, num_subcores=NS)

M_CH, M_K = 128, 40
W_CH, W_K = 128, 10


def _wid():
    return lax.axis_index("s") * NC + lax.axis_index("c")


def _add_rows(dst, src, rows):
    def body(r, carry):
        for c in range(LATENT // 16):
            sl = pl.ds(c * 16, 16)
            dst[r, sl] = dst[r, sl] + src[r, sl]
        return carry
    lax.fori_loop(0, rows, body, 0, unroll=2)


def _sc_gather_add(psm, pdm, psw, pdw, im_s, im_d, iw_s, iw_d):
    @functools.partial(
        pl.kernel, mesh=_MESH,
        compiler_params=pltpu.CompilerParams(use_tc_tiling_on_sc=False),
        out_type=[jax.ShapeDtypeStruct((EMP, LATENT), F32),
                  jax.ShapeDtypeStruct((EWP, LATENT), F32)],
        scratch_types=[
            pltpu.VMEM((M_K, M_CH), jnp.int32),
            pltpu.VMEM((M_K, M_CH), jnp.int32),
            pltpu.VMEM((W_K, W_CH), jnp.int32),
            pltpu.VMEM((W_K, W_CH), jnp.int32),
            pltpu.VMEM((128, LATENT), F32),
            pltpu.VMEM((128, LATENT), F32),
            pltpu.SemaphoreType.DMA,
        ],
    )
    def kfn(psm_h, pdm_h, psw_h, pdw_h, im_s_h, im_d_h, iw_s_h, iw_d_h,
            gm_h, gw_h, ia_v, ib_v, iwa_v, iwb_v, ba, bb, sem):
        w = _wid()
        pltpu.sync_copy(im_s_h.at[w], ia_v)
        pltpu.sync_copy(im_d_h.at[w], ib_v)
        pltpu.sync_copy(iw_s_h.at[w], iwa_v)
        pltpu.sync_copy(iw_d_h.at[w], iwb_v)

        base_m = w * (M_K * M_CH)
        base_w = w * (W_K * W_CH)

        def mesh_body(j, carry):
            d1 = pltpu.async_copy(psm_h.at[ia_v.at[j]], ba, sem)
            d2 = pltpu.async_copy(pdm_h.at[ib_v.at[j]], bb, sem)
            d1.wait()
            d2.wait()
            _add_rows(ba, bb, M_CH)
            pltpu.sync_copy(ba, gm_h.at[pl.ds(base_m + j * M_CH, M_CH)])
            return carry
        lax.fori_loop(0, M_K, mesh_body, 0)

        def world_body(j, carry):
            d1 = pltpu.async_copy(psw_h.at[iwa_v.at[j]], ba, sem)
            d2 = pltpu.async_copy(pdw_h.at[iwb_v.at[j]], bb, sem)
            d1.wait()
            d2.wait()
            _add_rows(ba, bb, W_CH)
            pltpu.sync_copy(ba, gw_h.at[pl.ds(base_w + j * W_CH, W_CH)])
            return carry
        lax.fori_loop(0, W_K, world_body, 0)

    return kfn(psm, pdm, psw, pdw, im_s, im_d, iw_s, iw_d)


def _sc_segment_sum(ne0, ne1, im_d, iw_d):
    n_per_tile = NP // NS

    @functools.partial(
        pl.kernel, mesh=_MESH,
        compiler_params=pltpu.CompilerParams(use_tc_tiling_on_sc=False),
        out_type=[jax.ShapeDtypeStruct((NP, LATENT), F32) for _ in range(4)],
        scratch_types=[
            pltpu.VMEM((M_K, M_CH), jnp.int32),
            pltpu.VMEM((W_K, W_CH), jnp.int32),
            pltpu.VMEM((128, LATENT), F32),
            pltpu.VMEM((128, LATENT), F32),
            pltpu.VMEM_SHARED((NP, LATENT), F32),
            pltpu.SemaphoreType.DMA,
        ],
    )
    def kfn(ne0_h, ne1_h, im_d_h, iw_d_h, o0a_h, o0b_h, o1a_h, o1b_h,
            idm_v, idw_v, dbuf, zbuf, acc, sem):
        c = lax.axis_index("c")
        s = lax.axis_index("s")
        w = _wid()
        pltpu.sync_copy(im_d_h.at[w], idm_v)
        pltpu.sync_copy(iw_d_h.at[w], idw_v)

        def zero_body(r, carry):
            for cc in range(LATENT // 16):
                zbuf[r, pl.ds(cc * 16, 16)] = jnp.zeros((16,), F32)
            return carry
        lax.fori_loop(0, 128, zero_body, 0, unroll=2)

        tile_base = s * n_per_tile

        def zero_acc():
            for t in range(5):
                pltpu.sync_copy(zbuf, acc.at[pl.ds(tile_base + t * 128, 128)])

        def readback(out_a, out_b):
            for t in range(5):
                sl = pl.ds(tile_base + t * 128, 128)
                pltpu.sync_copy(acc.at[sl], dbuf)

                @pl.when(c == 0)
                def _():
                    pltpu.sync_copy(dbuf, out_a.at[sl])

                @pl.when(c == 1)
                def _():
                    pltpu.sync_copy(dbuf, out_b.at[sl])

        zero_acc()
        plsc.subcore_barrier()

        base_m = w * (M_K * M_CH)

        def mesh_body(j, carry):
            pltpu.async_copy(
                ne0_h.at[pl.ds(base_m + j * M_CH, M_CH)], dbuf, sem).wait()
            pltpu.sync_copy(dbuf, acc.at[idm_v.at[j]], add=True)
            return carry
        lax.fori_loop(0, M_K, mesh_body, 0)

        plsc.subcore_barrier()
        readback(o0a_h, o0b_h)
        zero_acc()
        plsc.subcore_barrier()

        base_w = w * (W_K * W_CH)

        def world_body(j, carry):
            pltpu.async_copy(
                ne1_h.at[pl.ds(base_w + j * W_CH, W_CH)], dbuf, sem).wait()
            pltpu.sync_copy(dbuf, acc.at[idw_v.at[j]], add=True)
            return carry
        lax.fori_loop(0, W_K, world_body, 0)

        plsc.subcore_barrier()
        readback(o1a_h, o1b_h)

    return kfn(ne0, ne1, im_d, iw_d)


def _sc_pos_features(tsrc, tdst, tw, im_s, im_d, iw_s, iw_d):
    @functools.partial(
        pl.kernel, mesh=_MESH,
        compiler_params=pltpu.CompilerParams(use_tc_tiling_on_sc=False),
        out_type=[jax.ShapeDtypeStruct((EMP, 16), F32),
                  jax.ShapeDtypeStruct((EWP, 16), F32)],
        scratch_types=[
            pltpu.VMEM((M_K, M_CH), jnp.int32),
            pltpu.VMEM((M_K, M_CH), jnp.int32),
            pltpu.VMEM((W_K, W_CH), jnp.int32),
            pltpu.VMEM((W_K, W_CH), jnp.int32),
            pltpu.VMEM((128, 16), F32),
            pltpu.VMEM((128, 16), F32),
            pltpu.SemaphoreType.DMA,
        ],
    )
    def kfn(tsrc_h, tdst_h, tw_h, im_s_h, im_d_h, iw_s_h, iw_d_h,
            rm_h, rw_h, ia_v, ib_v, iwa_v, iwb_v, ba, bb, sem):
        w = _wid()
        pltpu.sync_copy(im_s_h.at[w], ia_v)
        pltpu.sync_copy(im_d_h.at[w], ib_v)
        pltpu.sync_copy(iw_s_h.at[w], iwa_v)
        pltpu.sync_copy(iw_d_h.at[w], iwb_v)

        def sub_rows(rows):
            def body(r, carry):
                sl = pl.ds(0, 16)
                ba[r, sl] = ba[r, sl] - bb[r, sl]
                return carry
            lax.fori_loop(0, rows, body, 0, unroll=4)

        base_m = w * (M_K * M_CH)

        def mesh_body(j, carry):
            d1 = pltpu.async_copy(tsrc_h.at[ia_v.at[j]], ba, sem)
            d2 = pltpu.async_copy(tdst_h.at[ib_v.at[j]], bb, sem)
            d1.wait()
            d2.wait()
            sub_rows(M_CH)
            pltpu.sync_copy(ba, rm_h.at[pl.ds(base_m + j * M_CH, M_CH)])
            return carry
        lax.fori_loop(0, M_K, mesh_body, 0)

        base_w = w * (W_K * W_CH)

        def world_body(j, carry):
            d1 = pltpu.async_copy(tw_h.at[iwa_v.at[j]], ba, sem)
            d2 = pltpu.async_copy(tw_h.at[iwb_v.at[j]], bb, sem)
            d1.wait()
            d2.wait()
            sub_rows(W_CH)
            pltpu.sync_copy(ba, rw_h.at[pl.ds(base_w + j * W_CH, W_CH)])
            return carry
        lax.fori_loop(0, W_K, world_body, 0)

    return kfn(tsrc, tdst, tw, im_s, im_d, iw_s, iw_d)


def _row(b):
    return b.reshape(1, -1)


def _fold_norm(w1, b1, mean, std):
    wn = w1 / std[:, None]
    b1f = b1 - (mean / std) @ w1
    return wn, b1f


def kernel(node_type, mesh_pos, world_pos, known_vel, srcs, dsts, wsrcs,
           wdsts, params):
    p = params

    n_w1, n_b1 = _fold_norm(p['node_enc']['W'][0], p['node_enc']['b'][0],
                            p['node_mean'], p['node_std'])
    n_w1p = jnp.zeros((16, LATENT), F32).at[:12].set(n_w1)

    m_w1, m_b1 = _fold_norm(p['mesh_enc']['W'][0], p['mesh_enc']['b'][0],
                            p['mesh_mean'], p['mesh_std'])
    m_w1p = jnp.zeros((16, LATENT), F32)
    m_w1p = m_w1p.at[0:3].set(m_w1[0:3]).at[3:6].set(m_w1[4:7])
    m_wn1 = _row(m_w1[3])
    m_wn2 = _row(m_w1[7])

    w_w1, w_b1 = _fold_norm(p['world_enc']['W'][0], p['world_enc']['b'][0],
                            p['world_mean'], p['world_std'])
    w_w1p = jnp.zeros((16, LATENT), F32).at[0:3].set(w_w1[0:3])
    w_wn1 = _row(w_w1[3])

    d_w3 = p['decoder']['W'][2] * p['out_std'][None, :]
    d_b3 = p['decoder']['b'][2] * p['out_std'] + p['out_mean']
    d_w3p = jnp.zeros((LATENT, LATENT), F32).at[:, :3].set(d_w3)
    d_b3p = jnp.zeros((LATENT,), F32).at[:3].set(d_b3)

    srcs_p = jnp.concatenate([srcs, jnp.zeros((EMP - EM,), jnp.int32)])
    dsts_p = jnp.concatenate([dsts, jnp.zeros((EMP - EM,), jnp.int32)])
    wsrcs_p = jnp.concatenate([wsrcs, jnp.zeros((EWP - EW,), jnp.int32)])
    wdsts_p = jnp.concatenate([wdsts, jnp.zeros((EWP - EW,), jnp.int32)])

    im_s = srcs_p.reshape(NW, M_K, M_CH)
    im_d = dsts_p.reshape(NW, M_K, M_CH)
    iw_s = wsrcs_p.reshape(NW, W_K, W_CH)
    iw_d = wdsts_p.reshape(NW, W_K, W_CH)

    padn = lambda a: jnp.concatenate(
        [a, jnp.zeros((NP - N, a.shape[1]), F32)], axis=0)
    pos_src = padn(jnp.concatenate(
        [mesh_pos, world_pos, jnp.zeros((N, 10), F32)], axis=1))
    pos_dst = padn(jnp.concatenate(
        [mesh_pos, mesh_pos, jnp.zeros((N, 10), F32)], axis=1))
    pos_w = padn(jnp.concatenate(
        [world_pos, jnp.zeros((N, 13), F32)], axis=1))
    rel8_m, rel8_w = _sc_pos_features(pos_src, pos_dst, pos_w,
                                      im_s, im_d, iw_s, iw_d)

    ne = p['node_enc']
    lat = _node_encoder(
        jnp.concatenate([node_type, jnp.zeros((NP - N,), jnp.int32)]).reshape(NP, 1),
        jnp.concatenate(
            [jnp.concatenate([known_vel, jnp.zeros((N, 5), F32)], axis=1),
             jnp.zeros((NP - N, 8), F32)], axis=0),
        n_w1p, _row(n_b1), ne['W'][1], _row(ne['b'][1]), ne['W'][2],
        _row(ne['b'][2]), _row(ne['ln_scale']), _row(ne['ln_bias']))

    me = p['mesh_enc']
    e0 = _edge_encoder(rel8_m, m_w1p, m_wn1, m_wn2, _row(m_b1), me['W'][1],
                       _row(me['b'][1]), me['W'][2], _row(me['b'][2]),
                       _row(me['ln_scale']), _row(me['ln_bias']),
                       two_norms=True, valid_rows=EM)
    we = p['world_enc']
    e1 = _edge_encoder(rel8_w, w_w1p, w_wn1, w_wn1, _row(w_b1), we['W'][1],
                       _row(we['b'][1]), we['W'][2], _row(we['b'][2]),
                       _row(we['ln_scale']), _row(we['ln_bias']),
                       two_norms=False, valid_rows=EW)

    def vcat_for(step):
        em, ew = p['edge_mp'][step]
        return jnp.concatenate(
            [em['W'][0][LATENT:2 * LATENT], em['W'][0][2 * LATENT:],
             ew['W'][0][LATENT:2 * LATENT], ew['W'][0][2 * LATENT:]], axis=1)

    psm, pdm, psw, pdw = _precompute(lat, vcat_for(0))
    for step in range(NUM_MP):
        em, ew = p['edge_mp'][step]
        nmp = p['node_mp'][step]

        gm, gw = _sc_gather_add(psm, pdm, psw, pdw, im_s, im_d, iw_s, iw_d)

        ne0 = _edge_mp(e0, gm, em['W'][0][:LATENT], _row(em['b'][0]),
                       em['W'][1], _row(em['b'][1]), em['W'][2],
                       _row(em['b'][2]), _row(em['ln_scale']),
                       _row(em['ln_bias']), valid_rows=EM)
        ne1 = _edge_mp(e1, gw, ew['W'][0][:LATENT], _row(ew['b'][0]),
                       ew['W'][1], _row(ew['b'][1]), ew['W'][2],
                       _row(ew['b'][2]), _row(ew['ln_scale']),
                       _row(ew['ln_bias']), valid_rows=EW)

        a0a, a0b, a1a, a1b = _sc_segment_sum(ne0, ne1, im_d, iw_d)

        vcat = (vcat_for(step + 1) if step + 1 < NUM_MP
                else jnp.zeros((LATENT, 4 * LATENT), F32))
        w1 = nmp['W'][0]
        lat, psm, pdm, psw, pdw = _node_mp(
            lat, a0a, a0b, a1a, a1b, w1[:LATENT], w1[LATENT:2 * LATENT],
            w1[2 * LATENT:], _row(nmp['b'][0]), nmp['W'][1],
            _row(nmp['b'][1]), nmp['W'][2], _row(nmp['b'][2]),
            _row(nmp['ln_scale']), _row(nmp['ln_bias']), vcat)
        e0, e1 = ne0, ne1

    dec = p['decoder']
    out = _decoder(lat, dec['W'][0], _row(dec['b'][0]), dec['W'][1],
                   _row(dec['b'][1]), d_w3p, _row(d_b3p))
    return out[:N, :3]
